# baseline (speedup 1.0000x reference)
import functools

import jax
import jax.numpy as jnp
from jax import lax
from jax.experimental import pallas as pl
from jax.experimental.pallas import tpu as pltpu
from jax.experimental.pallas import tpu_sc as plsc

_N = 5000
_NP = 5120
_B = 512
_NB = _NP // _B
_MAX_DET = 300
_MD_PAD = 384
_IOU_THRESH = 0.1

_NS = 8192
_LOG = 13

_SC_CORES = 2
_SC_SUBCORES = 16
_NW = _SC_CORES * _SC_SUBCORES
_BPW = _NP // _NW
_IDXW = 80
_NIDX = _BPW // _IDXW


def _bitonic_argsort(keys):
    pos = lax.broadcasted_iota(jnp.int32, (1, _NS), 1)

    def shl(x, d, fill):
        return jnp.concatenate(
            [x[:, d:], jnp.full((1, d), fill, x.dtype)], axis=1)

    def shr(x, d, fill):
        return jnp.concatenate(
            [jnp.full((1, d), fill, x.dtype), x[:, :_NS - d]], axis=1)

    k = keys
    v = pos
    for kk in range(_LOG):
        asc = (pos & (1 << (kk + 1))) == 0
        for j in range(kk, -1, -1):
            d = 1 << j
            i_low = (pos & d) == 0
            pk = jnp.where(i_low, shl(k, d, 0.0), shr(k, d, 0.0))
            pv = jnp.where(i_low, shl(v, d, 0), shr(v, d, 0))
            less = (k < pk) | ((k == pk) & (v < pv))
            min_k = jnp.where(less, k, pk)
            min_v = jnp.where(less, v, pv)
            max_k = jnp.where(less, pk, k)
            max_v = jnp.where(less, pv, v)
            sel_min = i_low == asc
            k = jnp.where(sel_min, min_k, max_k)
            v = jnp.where(sel_min, min_v, max_v)
    return v


def _decode_body(aT_ref, rT_ref, cT_ref, thr_ref, boxT_ref, s_ref, sm_ref,
                 argm_ref, order_ref, *, w_img, h_img):
    aT = aT_ref[...]
    rT = rT_ref[...]
    x1a = aT[0:1, :]
    y1a = aT[1:2, :]
    x2a = aT[2:3, :]
    y2a = aT[3:4, :]
    w = x2a - x1a
    h = y2a - y1a
    cx = x1a + 0.5 * w
    cy = y1a + 0.5 * h
    dx = rT[0:1, :] * 0.1
    dy = rT[1:2, :] * 0.1
    dw = rT[2:3, :] * 0.2
    dh = rT[3:4, :] * 0.2
    pcx = cx + dx * w
    pcy = cy + dy * h
    pw = jnp.exp(dw) * w
    ph = jnp.exp(dh) * h
    bx1 = jnp.maximum(pcx - 0.5 * pw, 0.0)
    by1 = jnp.maximum(pcy - 0.5 * ph, 0.0)
    bx2 = jnp.minimum(pcx + 0.5 * pw, w_img)
    by2 = jnp.minimum(pcy + 0.5 * ph, h_img)
    boxT_ref[...] = jnp.concatenate([bx1, by1, bx2, by2], axis=0)

    c = cT_ref[...]
    cmax = jnp.max(c, axis=0, keepdims=True)
    thr = thr_ref[0, 0]
    sm = jnp.where(cmax > thr, cmax, -jnp.inf)
    s_ref[...] = cmax
    sm_ref[...] = sm
    idxs = lax.broadcasted_iota(jnp.int32, c.shape, 0).astype(jnp.float32)
    cand = jnp.where(c == cmax, idxs, jnp.float32(2**30))
    argm_ref[...] = jnp.min(cand, axis=0, keepdims=True)

    keys = jnp.concatenate(
        [-sm, jnp.full((1, _NS - _NP), jnp.inf, jnp.float32)], axis=1)
    order_ref[...] = _bitonic_argsort(keys)


def _sc_gather_body(table_hbm, idx_hbm, out_hbm, idx_v, rows_v, sem):
    wid = lax.axis_index("s") * _SC_CORES + lax.axis_index("c")
    pltpu.sync_copy(idx_hbm.at[wid], idx_v)
    copies = [
        pltpu.async_copy(table_hbm.at[idx_v.at[j]],
                         rows_v.at[pl.ds(j * _IDXW, _IDXW)], sem)
        for j in range(_NIDX)
    ]
    for cp in copies:
        cp.wait()
    pltpu.sync_copy(rows_v, out_hbm.at[pl.ds(wid * _BPW, _BPW)])


def _sorted_gather(table, idx3):
    return pl.kernel(
        _sc_gather_body,
        out_type=jax.ShapeDtypeStruct((_NP, 8), jnp.float32),
        mesh=plsc.VectorSubcoreMesh(
            core_axis_name="c", subcore_axis_name="s",
            num_cores=_SC_CORES, num_subcores=_SC_SUBCORES),
        scratch_types=[
            pltpu.VMEM((_NIDX, _IDXW), jnp.int32),
            pltpu.VMEM((_BPW, 8), jnp.float32),
            pltpu.SemaphoreType.DMA,
        ],
        compiler_params=pltpu.CompilerParams(use_tc_tiling_on_sc=False),
    )(table, idx3)


def _lane_cumsum(x):
    s = 1
    while s < _NP:
        x = x + jnp.concatenate(
            [jnp.zeros((1, s), jnp.float32), x[:, :_NP - s]], axis=1)
        s *= 2
    return x


def _nms_body(tabT_ref, tab_ref, out_ref, m_ref, keep_ref):
    keep_ref[...] = jnp.ones((1, _NP), jnp.float32)

    def tile_mask(r0, c0):
        x1r = tab_ref[r0:r0 + _B, 0:1]
        y1r = tab_ref[r0:r0 + _B, 1:2]
        x2r = tab_ref[r0:r0 + _B, 2:3]
        y2r = tab_ref[r0:r0 + _B, 3:4]
        ar = (x2r - x1r) * (y2r - y1r)
        x1c = tabT_ref[0:1, c0:c0 + _B]
        y1c = tabT_ref[1:2, c0:c0 + _B]
        x2c = tabT_ref[2:3, c0:c0 + _B]
        y2c = tabT_ref[3:4, c0:c0 + _B]
        ac = (x2c - x1c) * (y2c - y1c)
        xx1 = jnp.maximum(x1r, x1c)
        yy1 = jnp.maximum(y1r, y1c)
        xx2 = jnp.minimum(x2r, x2c)
        yy2 = jnp.minimum(y2r, y2c)
        inter = (jnp.maximum(xx2 - xx1, 0.0) * jnp.maximum(yy2 - yy1, 0.0))
        iou = inter / (ar + ac - inter + 1e-8)
        return iou > _IOU_THRESH

    def suppress_from(k_bf16, m_bf16):
        prod = lax.dot_general(k_bf16, m_bf16, (((1,), (0,)), ((), ())),
                               preferred_element_type=jnp.float32)
        return prod

    for b in range(_NB):
        r0 = b * _B
        tri = (lax.broadcasted_iota(jnp.int32, (_B, _B), 0) <
               lax.broadcasted_iota(jnp.int32, (_B, _B), 1))
        wb = tile_mask(r0, r0) & tri
        m_ref[...] = jnp.where(wb, 1.0, 0.0).astype(jnp.bfloat16)
        kb0 = keep_ref[0:1, r0:r0 + _B]

        def w_cond(st):
            return st[1]

        def w_body(st):
            k, _ = st
            prod = suppress_from(k.astype(jnp.bfloat16), m_ref[...])
            knew = jnp.where(prod > 0, 0.0, kb0)
            return (knew, jnp.any(knew != k))

        kfin, _ = lax.while_loop(w_cond, w_body, (kb0, jnp.bool_(True)))
        keep_ref[0:1, r0:r0 + _B] = kfin

        k_bf = kfin.astype(jnp.bfloat16)
        for b2 in range(b + 1, _NB):
            c0 = b2 * _B
            mx = jnp.where(tile_mask(r0, c0), 1.0, 0.0).astype(jnp.bfloat16)
            prod = suppress_from(k_bf, mx)
            keep_ref[0:1, c0:c0 + _B] = (
                keep_ref[0:1, c0:c0 + _B] *
                jnp.where(prod > 0, 0.0, 1.0))

    ss = tabT_ref[4:5, :]
    keep = keep_ref[...] * jnp.where(ss > -jnp.inf, 1.0, 0.0)
    c1 = _lane_cumsum(keep)
    total_kept = c1[0:1, _NP - 1:_NP]
    pos1 = 1.0 + lax.broadcasted_iota(jnp.int32, (1, _NP), 1).astype(
        jnp.float32)
    c0 = pos1 - c1
    rank = jnp.where(keep > 0, c1 - 1.0, total_kept + c0 - 1.0)

    kio = lax.broadcasted_iota(jnp.int32, (_MD_PAD, _NP), 0).astype(
        jnp.float32)
    sel_mask = (rank == kio) & (kio < float(_MAX_DET))
    for ch in range(8):
        row = tabT_ref[ch:ch + 1, :]
        picked = jnp.where(sel_mask, row, 0.0)
        out_ref[:, ch:ch + 1] = jnp.sum(picked, axis=1, keepdims=True)


def kernel(img_batch, anchors, regression, classification, score_threshold):
    h_img = float(img_batch.shape[2])
    w_img = float(img_batch.shape[3])
    a = anchors[0]
    r = regression[0]
    c = classification[0]
    pad_n = _NP - _N
    aT = jnp.pad(a.T, ((0, 0), (0, pad_n)))
    rT = jnp.pad(r.T, ((0, 0), (0, pad_n)))
    cT = jnp.pad(c.T, ((0, 0), (0, pad_n)), constant_values=-jnp.inf)
    thr = jnp.reshape(score_threshold.astype(jnp.float32), (1, 1))

    boxT, s, sm, argm, order = pl.pallas_call(
        functools.partial(_decode_body, w_img=w_img, h_img=h_img),
        out_shape=[
            jax.ShapeDtypeStruct((4, _NP), jnp.float32),
            jax.ShapeDtypeStruct((1, _NP), jnp.float32),
            jax.ShapeDtypeStruct((1, _NP), jnp.float32),
            jax.ShapeDtypeStruct((1, _NP), jnp.float32),
            jax.ShapeDtypeStruct((1, _NS), jnp.int32),
        ],
    )(aT, rT, cT, thr)

    table = jnp.concatenate(
        [boxT.T, sm.T, s.T, argm.T,
         jnp.arange(_NP, dtype=jnp.float32).reshape(_NP, 1)], axis=1)
    idx3 = order[0, :_NP].reshape(_NW, _NIDX, _IDXW)
    sorted_tab = _sorted_gather(table, idx3)

    out = pl.pallas_call(
        _nms_body,
        out_shape=jax.ShapeDtypeStruct((_MD_PAD, 8), jnp.float32),
        scratch_shapes=[
            pltpu.VMEM((_B, _B), jnp.bfloat16),
            pltpu.VMEM((1, _NP), jnp.float32),
        ],
    )(sorted_tab.T, sorted_tab)

    nms_boxes = out[:_MAX_DET, 0:4]
    nms_scores = out[:_MAX_DET, 5]
    classes = out[:_MAX_DET, 6].astype(jnp.int32)
    sel = out[:_MAX_DET, 7].astype(jnp.int32)
    return nms_scores, classes, nms_boxes, sel


# speedup vs baseline: 1.2083x; 1.2083x over previous
import functools

import jax
import jax.numpy as jnp
from jax import lax
from jax.experimental import pallas as pl
from jax.experimental.pallas import tpu as pltpu
from jax.experimental.pallas import tpu_sc as plsc

_N = 5000
_NP = 5120
_B = 512
_NB = _NP // _B
_MAX_DET = 300
_MD_PAD = 384
_IOU_THRESH = 0.1

_SC_CORES = 2
_SC_SUBCORES = 16
_NW = _SC_CORES * _SC_SUBCORES
_BPW = _NP // _NW
_IDXW = 80
_NIDX = _BPW // _IDXW


def _decode_body(aT_ref, rT_ref, c_ref, thr_ref, table_ref, sm_ref,
                 *, w_img, h_img):
    aT = aT_ref[...]
    rT = rT_ref[...]
    x1a = aT[0:1, :]
    y1a = aT[1:2, :]
    x2a = aT[2:3, :]
    y2a = aT[3:4, :]
    w = x2a - x1a
    h = y2a - y1a
    cx = x1a + 0.5 * w
    cy = y1a + 0.5 * h
    dx = rT[0:1, :] * 0.1
    dy = rT[1:2, :] * 0.1
    dw = rT[2:3, :] * 0.2
    dh = rT[3:4, :] * 0.2
    pcx = cx + dx * w
    pcy = cy + dy * h
    pw = jnp.exp(dw) * w
    ph = jnp.exp(dh) * h
    bx1 = jnp.maximum(pcx - 0.5 * pw, 0.0)
    by1 = jnp.maximum(pcy - 0.5 * ph, 0.0)
    bx2 = jnp.minimum(pcx + 0.5 * pw, w_img)
    by2 = jnp.minimum(pcy + 0.5 * ph, h_img)
    boxT = jnp.concatenate([bx1, by1, bx2, by2], axis=0)

    c = c_ref[...]
    cmax = jnp.max(c, axis=1, keepdims=True)
    thr = thr_ref[0, 0]
    sm = jnp.where(cmax > thr, cmax, -jnp.inf)
    sm_ref[...] = sm
    idxs = lax.broadcasted_iota(jnp.int32, c.shape, 1)
    cand = jnp.where(c == cmax, jnp.float32(1.0) * idxs, jnp.float32(2**30))
    argm = jnp.min(cand, axis=1, keepdims=True)
    rowid = lax.broadcasted_iota(jnp.int32, (_NP, 1), 0).astype(jnp.float32)

    table_ref[...] = jnp.concatenate(
        [jnp.transpose(boxT, (1, 0)), sm, cmax, argm, rowid], axis=1)


def _sc_gather_body(table_hbm, idx_hbm, out_hbm, idx_v, rows_v, sem):
    wid = lax.axis_index("s") * _SC_CORES + lax.axis_index("c")
    pltpu.sync_copy(idx_hbm.at[wid], idx_v)
    copies = [
        pltpu.async_copy(table_hbm.at[idx_v.at[j]],
                         rows_v.at[pl.ds(j * _IDXW, _IDXW)], sem)
        for j in range(_NIDX)
    ]
    for cp in copies:
        cp.wait()
    pltpu.sync_copy(rows_v, out_hbm.at[pl.ds(wid * _BPW, _BPW)])


def _sorted_gather(table, idx3):
    return pl.kernel(
        _sc_gather_body,
        out_type=jax.ShapeDtypeStruct((_NP, 8), jnp.float32),
        mesh=plsc.VectorSubcoreMesh(
            core_axis_name="c", subcore_axis_name="s",
            num_cores=_SC_CORES, num_subcores=_SC_SUBCORES),
        scratch_types=[
            pltpu.VMEM((_NIDX, _IDXW), jnp.int32),
            pltpu.VMEM((_BPW, 8), jnp.float32),
            pltpu.SemaphoreType.DMA,
        ],
        compiler_params=pltpu.CompilerParams(use_tc_tiling_on_sc=False),
    )(table, idx3)


def _lane_cumsum(x):
    s = 1
    while s < _NP:
        x = x + jnp.concatenate(
            [jnp.zeros((1, s), jnp.float32), x[:, :_NP - s]], axis=1)
        s *= 2
    return x


def _nms_body(tab_ref, out_ref, m_ref, keep_ref):
    keep_ref[...] = jnp.ones((1, _NP), jnp.float32)
    tabT = jnp.transpose(tab_ref[...], (1, 0))

    def tile_mask(r0, c0):
        x1r = tab_ref[r0:r0 + _B, 0:1]
        y1r = tab_ref[r0:r0 + _B, 1:2]
        x2r = tab_ref[r0:r0 + _B, 2:3]
        y2r = tab_ref[r0:r0 + _B, 3:4]
        ar = (x2r - x1r) * (y2r - y1r)
        x1c = tabT[0:1, c0:c0 + _B]
        y1c = tabT[1:2, c0:c0 + _B]
        x2c = tabT[2:3, c0:c0 + _B]
        y2c = tabT[3:4, c0:c0 + _B]
        ac = (x2c - x1c) * (y2c - y1c)
        xx1 = jnp.maximum(x1r, x1c)
        yy1 = jnp.maximum(y1r, y1c)
        xx2 = jnp.minimum(x2r, x2c)
        yy2 = jnp.minimum(y2r, y2c)
        inter = (jnp.maximum(xx2 - xx1, 0.0) * jnp.maximum(yy2 - yy1, 0.0))
        iou = inter / (ar + ac - inter + 1e-8)
        return iou > _IOU_THRESH

    def suppress_from(k_bf16, m_bf16):
        prod = lax.dot_general(k_bf16, m_bf16, (((1,), (0,)), ((), ())),
                               preferred_element_type=jnp.float32)
        return prod

    for b in range(_NB):
        r0 = b * _B
        tri = (lax.broadcasted_iota(jnp.int32, (_B, _B), 0) <
               lax.broadcasted_iota(jnp.int32, (_B, _B), 1))
        wb = tile_mask(r0, r0) & tri
        m_ref[...] = jnp.where(wb, 1.0, 0.0).astype(jnp.bfloat16)
        kb0 = keep_ref[0:1, r0:r0 + _B]

        def w_cond(st):
            return st[1]

        def w_body(st):
            k, _ = st
            prod = suppress_from(k.astype(jnp.bfloat16), m_ref[...])
            knew = jnp.where(prod > 0, 0.0, kb0)
            return (knew, jnp.any(knew != k))

        kfin, _ = lax.while_loop(w_cond, w_body, (kb0, jnp.bool_(True)))
        keep_ref[0:1, r0:r0 + _B] = kfin

        k_bf = kfin.astype(jnp.bfloat16)
        for b2 in range(b + 1, _NB):
            c0 = b2 * _B
            mx = jnp.where(tile_mask(r0, c0), 1.0, 0.0).astype(jnp.bfloat16)
            prod = suppress_from(k_bf, mx)
            keep_ref[0:1, c0:c0 + _B] = (
                keep_ref[0:1, c0:c0 + _B] *
                jnp.where(prod > 0, 0.0, 1.0))

    ss = tabT[4:5, :]
    keep = keep_ref[...] * jnp.where(ss > -jnp.inf, 1.0, 0.0)
    c1 = _lane_cumsum(keep)
    total_kept = c1[0:1, _NP - 1:_NP]
    pos1 = 1.0 + lax.broadcasted_iota(jnp.int32, (1, _NP), 1).astype(
        jnp.float32)
    c0 = pos1 - c1
    rank = jnp.where(keep > 0, c1 - 1.0, total_kept + c0 - 1.0)

    kio = lax.broadcasted_iota(jnp.int32, (_MD_PAD, _NP), 0).astype(
        jnp.float32)
    sel_mask = (rank == kio) & (kio < float(_MAX_DET))
    for ch in range(8):
        row = tabT[ch:ch + 1, :]
        picked = jnp.where(sel_mask, row, 0.0)
        out_ref[:, ch:ch + 1] = jnp.sum(picked, axis=1, keepdims=True)


def kernel(img_batch, anchors, regression, classification, score_threshold):
    h_img = float(img_batch.shape[2])
    w_img = float(img_batch.shape[3])
    a = anchors[0]
    r = regression[0]
    c = classification[0]
    pad_n = _NP - _N
    aT = jnp.pad(a.T, ((0, 0), (0, pad_n)))
    rT = jnp.pad(r.T, ((0, 0), (0, pad_n)))
    cP = jnp.pad(c, ((0, pad_n), (0, 128 - c.shape[1])),
                 constant_values=-jnp.inf)
    thr = jnp.reshape(score_threshold.astype(jnp.float32), (1, 1))

    table, sm = pl.pallas_call(
        functools.partial(_decode_body, w_img=w_img, h_img=h_img),
        out_shape=[
            jax.ShapeDtypeStruct((_NP, 8), jnp.float32),
            jax.ShapeDtypeStruct((_NP, 1), jnp.float32),
        ],
    )(aT, rT, cP, thr)

    sm5 = sm[:_N, 0]
    order = jnp.argsort(-sm5)
    order_p = jnp.concatenate([order, jnp.arange(_N, _NP, dtype=order.dtype)])
    idx3 = order_p.reshape(_NW, _NIDX, _IDXW)
    sorted_tab = _sorted_gather(table, idx3)

    out = pl.pallas_call(
        _nms_body,
        out_shape=jax.ShapeDtypeStruct((_MD_PAD, 8), jnp.float32),
        scratch_shapes=[
            pltpu.VMEM((_B, _B), jnp.bfloat16),
            pltpu.VMEM((1, _NP), jnp.float32),
        ],
    )(sorted_tab)

    nms_boxes = out[:_MAX_DET, 0:4]
    nms_scores = out[:_MAX_DET, 5]
    classes = out[:_MAX_DET, 6].astype(jnp.int32)
    sel = out[:_MAX_DET, 7].astype(jnp.int32)
    return nms_scores, classes, nms_boxes, sel


# speedup vs baseline: 1.2332x; 1.0206x over previous
import functools

import jax
import jax.numpy as jnp
from jax import lax
from jax.experimental import pallas as pl
from jax.experimental.pallas import tpu as pltpu
from jax.experimental.pallas import tpu_sc as plsc

_N = 5000
_NP = 5120
_B = 512
_NB = _NP // _B
_MAX_DET = 300
_MD_PAD = 384
_IOU_THRESH = 0.1

_SC_CORES = 2
_SC_SUBCORES = 16
_NW = _SC_CORES * _SC_SUBCORES
_BPW = _NP // _NW
_IDXW = 80
_NIDX = _BPW // _IDXW


def _decode_body(aT_ref, rT_ref, c_ref, thr_ref, table_ref, sm_ref,
                 *, w_img, h_img):
    aT = aT_ref[...]
    rT = rT_ref[...]
    x1a = aT[0:1, :]
    y1a = aT[1:2, :]
    x2a = aT[2:3, :]
    y2a = aT[3:4, :]
    w = x2a - x1a
    h = y2a - y1a
    cx = x1a + 0.5 * w
    cy = y1a + 0.5 * h
    dx = rT[0:1, :] * 0.1
    dy = rT[1:2, :] * 0.1
    dw = rT[2:3, :] * 0.2
    dh = rT[3:4, :] * 0.2
    pcx = cx + dx * w
    pcy = cy + dy * h
    pw = jnp.exp(dw) * w
    ph = jnp.exp(dh) * h
    bx1 = jnp.maximum(pcx - 0.5 * pw, 0.0)
    by1 = jnp.maximum(pcy - 0.5 * ph, 0.0)
    bx2 = jnp.minimum(pcx + 0.5 * pw, w_img)
    by2 = jnp.minimum(pcy + 0.5 * ph, h_img)
    boxT = jnp.concatenate([bx1, by1, bx2, by2], axis=0)

    c = c_ref[...]
    cmax = jnp.max(c, axis=1, keepdims=True)
    thr = thr_ref[0, 0]
    sm = jnp.where(cmax > thr, cmax, -jnp.inf)
    sm_ref[0:_N, :] = sm
    sm_ref[_N:_NP, :] = jnp.full((_NP - _N, 1), -jnp.inf, jnp.float32)
    idxs = lax.broadcasted_iota(jnp.int32, c.shape, 1)
    cand = jnp.where(c == cmax, jnp.float32(1.0) * idxs, jnp.float32(2**30))
    argm = jnp.min(cand, axis=1, keepdims=True)
    rowid = lax.broadcasted_iota(jnp.int32, (_N, 1), 0).astype(jnp.float32)

    table_ref[0:_N, :] = jnp.concatenate(
        [jnp.transpose(boxT, (1, 0)), sm, cmax, argm, rowid], axis=1)
    pad_rows = jnp.concatenate(
        [jnp.zeros((_NP - _N, 4), jnp.float32),
         jnp.full((_NP - _N, 2), -jnp.inf, jnp.float32),
         jnp.zeros((_NP - _N, 1), jnp.float32),
         jnp.float32(_N) + lax.broadcasted_iota(
             jnp.int32, (_NP - _N, 1), 0).astype(jnp.float32)], axis=1)
    table_ref[_N:_NP, :] = pad_rows


def _sc_gather_body(table_hbm, idx_hbm, out_hbm, idx_v, rows_v, sem):
    wid = lax.axis_index("s") * _SC_CORES + lax.axis_index("c")
    pltpu.sync_copy(idx_hbm.at[wid], idx_v)
    copies = [
        pltpu.async_copy(table_hbm.at[idx_v.at[j]],
                         rows_v.at[pl.ds(j * _IDXW, _IDXW)], sem)
        for j in range(_NIDX)
    ]
    for cp in copies:
        cp.wait()
    pltpu.sync_copy(rows_v, out_hbm.at[pl.ds(wid * _BPW, _BPW)])


def _sorted_gather(table, idx3):
    return pl.kernel(
        _sc_gather_body,
        out_type=jax.ShapeDtypeStruct((_NP, 8), jnp.float32),
        mesh=plsc.VectorSubcoreMesh(
            core_axis_name="c", subcore_axis_name="s",
            num_cores=_SC_CORES, num_subcores=_SC_SUBCORES),
        scratch_types=[
            pltpu.VMEM((_NIDX, _IDXW), jnp.int32),
            pltpu.VMEM((_BPW, 8), jnp.float32),
            pltpu.SemaphoreType.DMA,
        ],
        compiler_params=pltpu.CompilerParams(use_tc_tiling_on_sc=False),
    )(table, idx3)


def _lane_cumsum(x):
    s = 1
    while s < _NP:
        x = x + jnp.concatenate(
            [jnp.zeros((1, s), jnp.float32), x[:, :_NP - s]], axis=1)
        s *= 2
    return x


def _nms_body(tab_ref, out_ref, m_ref, keep_ref):
    keep_ref[...] = jnp.ones((1, _NP), jnp.float32)
    tabT = jnp.transpose(tab_ref[...], (1, 0))

    def tile_mask(r0, c0):
        x1r = tab_ref[r0:r0 + _B, 0:1]
        y1r = tab_ref[r0:r0 + _B, 1:2]
        x2r = tab_ref[r0:r0 + _B, 2:3]
        y2r = tab_ref[r0:r0 + _B, 3:4]
        ar = (x2r - x1r) * (y2r - y1r)
        x1c = tabT[0:1, c0:c0 + _B]
        y1c = tabT[1:2, c0:c0 + _B]
        x2c = tabT[2:3, c0:c0 + _B]
        y2c = tabT[3:4, c0:c0 + _B]
        ac = (x2c - x1c) * (y2c - y1c)
        xx1 = jnp.maximum(x1r, x1c)
        yy1 = jnp.maximum(y1r, y1c)
        xx2 = jnp.minimum(x2r, x2c)
        yy2 = jnp.minimum(y2r, y2c)
        inter = (jnp.maximum(xx2 - xx1, 0.0) * jnp.maximum(yy2 - yy1, 0.0))
        iou = inter / (ar + ac - inter + 1e-8)
        return iou > _IOU_THRESH

    def suppress_from(k_bf16, m_bf16):
        prod = lax.dot_general(k_bf16, m_bf16, (((1,), (0,)), ((), ())),
                               preferred_element_type=jnp.float32)
        return prod

    for b in range(_NB):
        r0 = b * _B
        tri = (lax.broadcasted_iota(jnp.int32, (_B, _B), 0) <
               lax.broadcasted_iota(jnp.int32, (_B, _B), 1))
        wb = tile_mask(r0, r0) & tri
        m_ref[...] = jnp.where(wb, 1.0, 0.0).astype(jnp.bfloat16)
        kb0 = keep_ref[0:1, r0:r0 + _B]

        def w_cond(st):
            return st[1]

        def w_body(st):
            k, _ = st
            prod = suppress_from(k.astype(jnp.bfloat16), m_ref[...])
            knew = jnp.where(prod > 0, 0.0, kb0)
            return (knew, jnp.any(knew != k))

        kfin, _ = lax.while_loop(w_cond, w_body, (kb0, jnp.bool_(True)))
        keep_ref[0:1, r0:r0 + _B] = kfin

        k_bf = kfin.astype(jnp.bfloat16)
        for b2 in range(b + 1, _NB):
            c0 = b2 * _B
            mx = jnp.where(tile_mask(r0, c0), 1.0, 0.0).astype(jnp.bfloat16)
            prod = suppress_from(k_bf, mx)
            keep_ref[0:1, c0:c0 + _B] = (
                keep_ref[0:1, c0:c0 + _B] *
                jnp.where(prod > 0, 0.0, 1.0))

    ss = tabT[4:5, :]
    keep = keep_ref[...] * jnp.where(ss > -jnp.inf, 1.0, 0.0)
    c1 = _lane_cumsum(keep)
    total_kept = c1[0:1, _NP - 1:_NP]
    pos1 = 1.0 + lax.broadcasted_iota(jnp.int32, (1, _NP), 1).astype(
        jnp.float32)
    c0 = pos1 - c1
    rank = jnp.where(keep > 0, c1 - 1.0, total_kept + c0 - 1.0)

    kio = lax.broadcasted_iota(jnp.int32, (_MD_PAD, _NP), 0).astype(
        jnp.float32)
    sel_mask = (rank == kio) & (kio < float(_MAX_DET))
    for ch in range(8):
        row = tabT[ch:ch + 1, :]
        picked = jnp.where(sel_mask, row, 0.0)
        out_ref[:, ch:ch + 1] = jnp.sum(picked, axis=1, keepdims=True)


def kernel(img_batch, anchors, regression, classification, score_threshold):
    h_img = float(img_batch.shape[2])
    w_img = float(img_batch.shape[3])
    a = anchors[0]
    r = regression[0]
    c = classification[0]
    pad_n = _NP - _N
    thr = jnp.reshape(score_threshold.astype(jnp.float32), (1, 1))

    table, sm = pl.pallas_call(
        functools.partial(_decode_body, w_img=w_img, h_img=h_img),
        out_shape=[
            jax.ShapeDtypeStruct((_NP, 8), jnp.float32),
            jax.ShapeDtypeStruct((_NP, 1), jnp.float32),
        ],
    )(a.T, r.T, c, thr)

    sm5 = sm[:_N, 0]
    order = jnp.argsort(-sm5)
    order_p = jnp.concatenate([order, jnp.arange(_N, _NP, dtype=order.dtype)])
    idx3 = order_p.reshape(_NW, _NIDX, _IDXW)
    sorted_tab = _sorted_gather(table, idx3)

    out = pl.pallas_call(
        _nms_body,
        out_shape=jax.ShapeDtypeStruct((_MD_PAD, 8), jnp.float32),
        scratch_shapes=[
            pltpu.VMEM((_B, _B), jnp.bfloat16),
            pltpu.VMEM((1, _NP), jnp.float32),
        ],
    )(sorted_tab)

    nms_boxes = out[:_MAX_DET, 0:4]
    nms_scores = out[:_MAX_DET, 5]
    classes = out[:_MAX_DET, 6].astype(jnp.int32)
    sel = out[:_MAX_DET, 7].astype(jnp.int32)
    return nms_scores, classes, nms_boxes, sel
